# baseline (speedup 1.0000x reference)
import jax
import jax.numpy as jnp
from jax.experimental import pallas as pl
from jax.experimental.pallas import tpu as pltpu

_TN = 1024


def _moe_kernel(x_ref, xb_ref, wr_ref, br_ref, we_ref, be_ref, out_ref,
                gates_ref):
    e = pl.program_id(1)
    n_e = pl.num_programs(1)
    E = gates_ref.shape[1]

    @pl.when(e == 0)
    def _compute_gates():
        logits = jnp.dot(x_ref[...], wr_ref[...],
                         preferred_element_type=jnp.float32)
        logits = logits + br_ref[...]
        m = jnp.max(logits, axis=1, keepdims=True)
        ex = jnp.exp(logits - m)
        gates_ref[...] = ex / jnp.sum(ex, axis=1, keepdims=True)

    y = jnp.dot(xb_ref[...], we_ref[0], preferred_element_type=jnp.float32)
    g = gates_ref[...]
    eidx = jax.lax.broadcasted_iota(jnp.int32, (1, E), 1)
    ge = jnp.sum(jnp.where(eidx == e, g, 0.0), axis=1, keepdims=True)
    contrib = y * ge

    @pl.when(e == 0)
    def _init():
        out_ref[...] = contrib

    @pl.when(e > 0)
    def _acc():
        out_ref[...] += contrib

    @pl.when(e == n_e - 1)
    def _bias():
        out_ref[...] += jnp.dot(g, be_ref[...],
                                preferred_element_type=jnp.float32)


def kernel(x, Wr, br, We, be):
    N, D = x.shape
    E = We.shape[0]
    br2 = br.reshape(1, E)
    xb = x.astype(jnp.bfloat16)
    web = We.astype(jnp.bfloat16)
    return pl.pallas_call(
        _moe_kernel,
        grid=(N // _TN, E),
        in_specs=[
            pl.BlockSpec((_TN, D), lambda i, e: (i, 0)),
            pl.BlockSpec((_TN, D), lambda i, e: (i, 0)),
            pl.BlockSpec((D, E), lambda i, e: (0, 0)),
            pl.BlockSpec((1, E), lambda i, e: (0, 0)),
            pl.BlockSpec((1, D, D), lambda i, e: (e, 0, 0)),
            pl.BlockSpec((E, D), lambda i, e: (0, 0)),
        ],
        out_specs=pl.BlockSpec((_TN, D), lambda i, e: (i, 0)),
        out_shape=jax.ShapeDtypeStruct((N, D), jnp.float32),
        scratch_shapes=[pltpu.VMEM((_TN, E), jnp.float32)],
        compiler_params=pltpu.CompilerParams(
            dimension_semantics=("parallel", "arbitrary")),
    )(x, xb, Wr, br2, web, be)


# speedup vs baseline: 1.2212x; 1.2212x over previous
import jax
import jax.numpy as jnp
from jax.experimental import pallas as pl
from jax.experimental.pallas import tpu as pltpu

_TN = 1024


def _moe_kernel(x_ref, wr_ref, br_ref, we_ref, be_ref, out_ref, gates_ref):
    e = pl.program_id(0)
    n_e = pl.num_programs(0)
    N = x_ref.shape[0]
    E = gates_ref.shape[1]

    @pl.when(e == 0)
    def _compute_gates():
        logits = jnp.dot(x_ref[...], wr_ref[...],
                         preferred_element_type=jnp.float32)
        logits = logits + br_ref[...]
        m = jnp.max(logits, axis=1, keepdims=True)
        ex = jnp.exp(logits - m)
        gates_ref[...] = ex / jnp.sum(ex, axis=1, keepdims=True)

    g = gates_ref[...]
    eidx = jax.lax.broadcasted_iota(jnp.int32, (1, E), 1)
    ge = jnp.sum(jnp.where(eidx == e, g, 0.0), axis=1, keepdims=True)

    for t in range(N // _TN):
        rows = slice(t * _TN, (t + 1) * _TN)
        y = jnp.dot(x_ref[rows, :], we_ref[0],
                    preferred_element_type=jnp.float32)
        contrib = y * ge[rows, :]

        @pl.when(e == 0)
        def _init():
            out_ref[rows, :] = contrib

        @pl.when(e > 0)
        def _acc():
            out_ref[rows, :] += contrib

    @pl.when(e == n_e - 1)
    def _bias():
        out_ref[...] += jnp.dot(g, be_ref[...],
                                preferred_element_type=jnp.float32)


def kernel(x, Wr, br, We, be):
    N, D = x.shape
    E = We.shape[0]
    br2 = br.reshape(1, E)
    return pl.pallas_call(
        _moe_kernel,
        grid=(E,),
        in_specs=[
            pl.BlockSpec((N, D), lambda e: (0, 0)),
            pl.BlockSpec((D, E), lambda e: (0, 0)),
            pl.BlockSpec((1, E), lambda e: (0, 0)),
            pl.BlockSpec((1, D, D), lambda e: (e, 0, 0)),
            pl.BlockSpec((E, D), lambda e: (0, 0)),
        ],
        out_specs=pl.BlockSpec((N, D), lambda e: (0, 0)),
        out_shape=jax.ShapeDtypeStruct((N, D), jnp.float32),
        scratch_shapes=[pltpu.VMEM((N, E), jnp.float32)],
        compiler_params=pltpu.CompilerParams(
            dimension_semantics=("arbitrary",)),
    )(x, Wr, br2, We, be)


# speedup vs baseline: 1.2623x; 1.0336x over previous
import jax
import jax.numpy as jnp
from jax.experimental import pallas as pl
from jax.experimental.pallas import tpu as pltpu

_TN = 256


def _moe_kernel(x_ref, wr_ref, br_ref, w2_ref, be_ref, out_ref, a_ref):
    xv = x_ref[...]
    D = xv.shape[1]

    logits = jnp.dot(xv, wr_ref[...], preferred_element_type=jnp.float32)
    logits = logits + br_ref[...]
    m = jnp.max(logits, axis=1, keepdims=True)
    ex = jnp.exp(logits - m)
    g = ex / jnp.sum(ex, axis=1, keepdims=True)
    E = g.shape[1]

    for e in range(E):
        a_ref[:, e * D:(e + 1) * D] = xv * g[:, e:e + 1]

    y = jnp.dot(a_ref[...], w2_ref[...], preferred_element_type=jnp.float32)
    out_ref[...] = y + jnp.dot(g, be_ref[...],
                               preferred_element_type=jnp.float32)


def kernel(x, Wr, br, We, be):
    N, D = x.shape
    E = We.shape[0]
    br2 = br.reshape(1, E)
    W2 = We.reshape(E * D, D)
    return pl.pallas_call(
        _moe_kernel,
        grid=(N // _TN,),
        in_specs=[
            pl.BlockSpec((_TN, D), lambda i: (i, 0)),
            pl.BlockSpec((D, E), lambda i: (0, 0)),
            pl.BlockSpec((1, E), lambda i: (0, 0)),
            pl.BlockSpec((E * D, D), lambda i: (0, 0)),
            pl.BlockSpec((E, D), lambda i: (0, 0)),
        ],
        out_specs=pl.BlockSpec((_TN, D), lambda i: (i, 0)),
        out_shape=jax.ShapeDtypeStruct((N, D), jnp.float32),
        scratch_shapes=[pltpu.VMEM((_TN, E * D), jnp.float32)],
        compiler_params=pltpu.CompilerParams(
            dimension_semantics=("arbitrary",)),
    )(x, Wr, br2, W2, be)


# speedup vs baseline: 1.3634x; 1.0801x over previous
import jax
import jax.numpy as jnp
from jax.experimental import pallas as pl
from jax.experimental.pallas import tpu as pltpu

_TN = 512


def _moe_kernel(x_ref, wr_ref, br_ref, w2_ref, be_ref, out_ref, a_ref):
    xv = x_ref[...]
    D = xv.shape[1]

    logits = jnp.dot(xv, wr_ref[...], preferred_element_type=jnp.float32)
    logits = logits + br_ref[...]
    m = jnp.max(logits, axis=1, keepdims=True)
    ex = jnp.exp(logits - m)
    g = ex / jnp.sum(ex, axis=1, keepdims=True)
    E = g.shape[1]

    for e in range(E):
        a_ref[:, e * D:(e + 1) * D] = xv * g[:, e:e + 1]

    y = jnp.dot(a_ref[...], w2_ref[...], preferred_element_type=jnp.float32)
    out_ref[...] = y + jnp.dot(g, be_ref[...],
                               preferred_element_type=jnp.float32)


def kernel(x, Wr, br, We, be):
    N, D = x.shape
    E = We.shape[0]
    br2 = br.reshape(1, E)
    W2 = We.reshape(E * D, D)
    return pl.pallas_call(
        _moe_kernel,
        grid=(N // _TN,),
        in_specs=[
            pl.BlockSpec((_TN, D), lambda i: (i, 0)),
            pl.BlockSpec((D, E), lambda i: (0, 0)),
            pl.BlockSpec((1, E), lambda i: (0, 0)),
            pl.BlockSpec((E * D, D), lambda i: (0, 0)),
            pl.BlockSpec((E, D), lambda i: (0, 0)),
        ],
        out_specs=pl.BlockSpec((_TN, D), lambda i: (i, 0)),
        out_shape=jax.ShapeDtypeStruct((N, D), jnp.float32),
        scratch_shapes=[pltpu.VMEM((_TN, E * D), jnp.float32)],
        compiler_params=pltpu.CompilerParams(
            dimension_semantics=("arbitrary",)),
    )(x, Wr, br2, W2, be)
